# Initial kernel scaffold; baseline (speedup 1.0000x reference)
import jax
import jax.numpy as jnp
from jax.experimental import pallas as pl


def kernel(x, edge_index, edge_weight, W_enc, b_enc, W_gcn, b_gcn, W_dec, b_dec):
    raise NotImplementedError("write your pallas kernel here")


# speedup vs baseline: 27.6862x; 27.6862x over previous
import functools

import jax
import jax.numpy as jnp
from jax import lax
from jax.experimental import pallas as pl
from jax.experimental.pallas import tpu as pltpu
from jax.experimental.pallas import tpu_sc as plsc

N_NODES = 10000
N_EDGES = 320000
IN_FEATS = 128
HIDDEN = 32

N_TILES = 32
EPT = N_EDGES // N_TILES
N_GROUPS = 4
F_TILE = HIDDEN // (N_TILES // N_GROUPS)
EPG = N_EDGES // N_GROUPS
CHUNK = 8000

_SC_MESH = plsc.VectorSubcoreMesh(core_axis_name="c", subcore_axis_name="s")
_SC_PARAMS = pltpu.CompilerParams(needs_layout_passes=False)


@functools.partial(
    pl.kernel,
    out_type=jax.ShapeDtypeStruct((N_TILES, N_NODES), jnp.float32),
    mesh=_SC_MESH,
    scratch_types=[
        pltpu.VMEM((N_NODES,), jnp.float32),
        pltpu.VMEM((EPT,), jnp.int32),
        pltpu.VMEM((EPT,), jnp.float32),
    ],
    compiler_params=_SC_PARAMS,
)
def _deg_pass(col_hbm, ew_hbm, out_hbm, deg_v, col_v, ew_v):
    wid = lax.axis_index("s") * 2 + lax.axis_index("c")
    base = wid * EPT
    pltpu.sync_copy(col_hbm.at[pl.ds(base, EPT)], col_v)
    pltpu.sync_copy(ew_hbm.at[pl.ds(base, EPT)], ew_v)

    zero16 = jnp.zeros((16,), jnp.float32)

    def zero_body(i, c):
        deg_v[pl.ds(i * 16, 16)] = zero16
        return c

    lax.fori_loop(0, N_NODES // 16, zero_body, 0)

    def edge_body(i, c):
        c16 = col_v[pl.ds(i * 16, 16)]
        w16 = ew_v[pl.ds(i * 16, 16)]
        plsc.addupdate_scatter(deg_v, [c16], w16)
        return c

    lax.fori_loop(0, EPT // 16, edge_body, 0)
    pltpu.sync_copy(deg_v, out_hbm.at[wid])


@functools.partial(
    pl.kernel,
    out_type=jax.ShapeDtypeStruct((N_GROUPS, HIDDEN, N_NODES), jnp.float32),
    mesh=_SC_MESH,
    scratch_types=[
        pltpu.VMEM((F_TILE, N_NODES), jnp.float32),
        pltpu.VMEM((F_TILE, N_NODES), jnp.float32),
        pltpu.VMEM((CHUNK,), jnp.int32),
        pltpu.VMEM((CHUNK,), jnp.float32),
    ],
    compiler_params=_SC_PARAMS,
)
def _msg_pass(rc_hbm, ew_hbm, gt_hbm, out_hbm, g_v, acc_v, rc_v, ew_v):
    wid = lax.axis_index("s") * 2 + lax.axis_index("c")
    grp = lax.shift_right_logical(wid, 3)
    fb = wid & 7

    pltpu.sync_copy(gt_hbm.at[pl.ds(fb * F_TILE, F_TILE)], g_v)

    zero16 = jnp.zeros((16,), jnp.float32)

    def zero_body(i, c):
        for f in range(F_TILE):
            acc_v[f, pl.ds(i * 16, 16)] = zero16
        return c

    lax.fori_loop(0, N_NODES // 16, zero_body, 0)

    fvecs = [jnp.full((16,), f, jnp.int32) for f in range(F_TILE)]

    def edge_body(i, c):
        rc16 = rc_v[pl.ds(i * 16, 16)]
        w16 = ew_v[pl.ds(i * 16, 16)]
        row = lax.shift_right_logical(rc16, 14)
        col = rc16 & 16383
        for f in range(F_TILE):
            vals = plsc.load_gather(g_v, [fvecs[f], row])
            plsc.addupdate_scatter(acc_v, [fvecs[f], col], vals * w16)
        return c

    def chunk_body(ci, c):
        off = grp * EPG + ci * CHUNK
        pltpu.sync_copy(rc_hbm.at[pl.ds(off, CHUNK)], rc_v)
        pltpu.sync_copy(ew_hbm.at[pl.ds(off, CHUNK)], ew_v)
        lax.fori_loop(0, CHUNK // 16, edge_body, 0)
        return c

    lax.fori_loop(0, EPG // CHUNK, chunk_body, 0)
    pltpu.sync_copy(acc_v, out_hbm.at[grp, pl.ds(fb * F_TILE, F_TILE)])


def _enc_body(x_ref, we_ref, be_ref, wg_ref, er_ref, ec_ref, hwt_ref, rc_ref):
    x = x_ref[...]
    xf = jnp.where(jnp.isnan(x), jnp.float32(0.0), x)
    h = lax.dot_general(xf, we_ref[...], (((1,), (1,)), ((), ())),
                        preferred_element_type=jnp.float32)
    h = jnp.maximum(h + be_ref[...], 0.0)
    hwt_ref[...] = lax.dot_general(
        wg_ref[...], h, (((1,), (1,)), ((), ())),
        preferred_element_type=jnp.float32)
    rc_ref[...] = er_ref[...] * 16384 + ec_ref[...]


_enc_pass = pl.pallas_call(
    _enc_body,
    out_shape=(
        jax.ShapeDtypeStruct((HIDDEN, N_NODES), jnp.float32),
        jax.ShapeDtypeStruct((N_EDGES // 128, 128), jnp.int32),
    ),
)


def _dis_body(degp_ref, hwt_ref, dis_ref, gt_ref):
    deg = 1.0 + jnp.sum(degp_ref[...], axis=0, keepdims=True)
    dis = jnp.where(deg > 0, lax.rsqrt(deg), 0.0)
    dis_ref[...] = dis
    gt_ref[...] = dis * hwt_ref[...]


_dis_pass = pl.pallas_call(
    _dis_body,
    out_shape=(
        jax.ShapeDtypeStruct((1, N_NODES), jnp.float32),
        jax.ShapeDtypeStruct((HIDDEN, N_NODES), jnp.float32),
    ),
)


def _dec_body(accp_ref, gt_ref, dis_ref, bg_ref, wd_ref, bd_ref, out_ref):
    accsum = jnp.sum(accp_ref[...], axis=0)
    pre = dis_ref[...] * (accsum + gt_ref[...]) + bg_ref[...]
    agg = jnp.maximum(pre, 0.0)
    rec = lax.dot_general(agg, wd_ref[...], (((0,), (1,)), ((), ())),
                          preferred_element_type=jnp.float32)
    out_ref[...] = rec + bd_ref[...]


_dec_pass = pl.pallas_call(
    _dec_body,
    out_shape=jax.ShapeDtypeStruct((N_NODES, IN_FEATS), jnp.float32),
)


def kernel(x, edge_index, edge_weight, W_enc, b_enc, W_gcn, b_gcn, W_dec, b_dec):
    ei = edge_index.astype(jnp.int32)
    er = ei[0].reshape(N_EDGES // 128, 128)
    ec = ei[1].reshape(N_EDGES // 128, 128)
    ew = edge_weight.astype(jnp.float32)

    deg_parts = _deg_pass(ei[1], ew)
    hwt, rc = _enc_pass(x, W_enc, b_enc.reshape(1, HIDDEN), W_gcn, er, ec)
    dis, gt = _dis_pass(deg_parts, hwt)
    acc_parts = _msg_pass(rc.reshape(N_EDGES), ew, gt)
    recon = _dec_pass(acc_parts, gt, dis, b_gcn.reshape(HIDDEN, 1),
                      W_dec, b_dec.reshape(1, IN_FEATS))
    return recon
